# baseline (device time: 21684 ns/iter reference)
import jax
import jax.numpy as jnp
from jax import lax
from jax.experimental import pallas as pl
from jax.experimental.pallas import tpu as pltpu

N_DEV = 4
B, SQ, SKV, DH, D_MODEL = 2, 128, 128, 64, 512
H_LOC = 4


def kernel(x, Wq, K_ext, V_ext, Wo):
    x2 = x.reshape(B * SQ, D_MODEL)

    def body(x_ref, wq_ref, k_ref, v_ref, wo_ref, out_ref,
             k_loc, v_loc, copy_sems, comm_ref, send_sems, recv_sems):
        my_pos = lax.axis_index("i")

        kv_copies = []
        for t, (hbm, vmem) in enumerate(((k_ref, k_loc), (v_ref, v_loc))):
            for b in range(B):
                for h in range(H_LOC):
                    c = pltpu.make_async_copy(
                        hbm.at[b, :, my_pos * H_LOC + h, :],
                        vmem.at[b, h],
                        copy_sems.at[t, b, h],
                    )
                    c.start()
                    kv_copies.append(c)
        xp = my_pos ^ 1
        yp = 3 - my_pos

        barrier_sem = pltpu.get_barrier_semaphore()
        for nbr in (xp, yp):
            pl.semaphore_signal(
                barrier_sem, inc=1,
                device_id=(nbr,), device_id_type=pl.DeviceIdType.MESH,
            )
        pl.semaphore_wait(barrier_sem, 2)

        def exchange(src_slot, dst_slot, sem_idx, partner):
            return pltpu.make_async_remote_copy(
                src_ref=comm_ref.at[src_slot],
                dst_ref=comm_ref.at[dst_slot],
                send_sem=send_sems.at[sem_idx],
                recv_sem=recv_sems.at[sem_idx],
                device_id=(partner,),
                device_id_type=pl.DeviceIdType.MESH,
            )

        q2 = jnp.dot(x_ref[:], wq_ref[:],
                     preferred_element_type=jnp.float32)

        rdmas = []
        for b in range(B):
            ctxs = []
            for h in range(H_LOC):
                qbh = q2[b * SQ:(b + 1) * SQ, h * DH:(h + 1) * DH]
                kv_copies[b * H_LOC + h].wait()
                kv_copies[8 + b * H_LOC + h].wait()
                kbh = k_loc[b, h]
                vbh = v_loc[b, h]
                s = lax.dot_general(
                    qbh, kbh, (((1,), (1,)), ((), ())),
                    preferred_element_type=jnp.float32,
                ) * 0.125
                m = jnp.max(s, axis=-1, keepdims=True)
                e = jnp.exp(s - m)
                w = e / jnp.sum(e, axis=-1, keepdims=True)
                ctxs.append(jnp.dot(w, vbh,
                                    preferred_element_type=jnp.float32))
            ctx_b = jnp.concatenate(ctxs, axis=-1)
            comm_ref[4 * b] = jnp.dot(ctx_b, wo_ref[:],
                                      preferred_element_type=jnp.float32)
            r = exchange(4 * b, 4 * b + 1, b, xp if b == 0 else yp)
            r.start()
            rdmas.append(r)

        for b in range(B):
            rdmas[b].wait_recv()
            comm_ref[4 * b + 2] = comm_ref[4 * b] + comm_ref[4 * b + 1]
            r = exchange(4 * b + 2, 4 * b + 3, 2 + b, yp if b == 0 else xp)
            r.start()
            rdmas.append(r)

        for b in range(B):
            rdmas[2 + b].wait_recv()
            out_ref[b] = comm_ref[4 * b + 2] + comm_ref[4 * b + 3]

        for r in rdmas:
            r.wait_send()

    return pl.pallas_call(
        body,
        out_shape=jax.ShapeDtypeStruct((B, SQ, D_MODEL), jnp.float32),
        in_specs=[
            pl.BlockSpec(memory_space=pltpu.VMEM),
            pl.BlockSpec(memory_space=pltpu.VMEM),
            pl.BlockSpec(memory_space=pltpu.MemorySpace.HBM),
            pl.BlockSpec(memory_space=pltpu.MemorySpace.HBM),
            pl.BlockSpec(memory_space=pltpu.VMEM),
        ],
        out_specs=pl.BlockSpec(memory_space=pltpu.VMEM),
        scratch_shapes=[
            pltpu.VMEM((B, H_LOC, SKV, DH), jnp.float32),
            pltpu.VMEM((B, H_LOC, SKV, DH), jnp.float32),
            pltpu.SemaphoreType.DMA((2, B, H_LOC)),
            pltpu.VMEM((8, SQ, D_MODEL), jnp.float32),
            pltpu.SemaphoreType.DMA((4,)),
            pltpu.SemaphoreType.DMA((4,)),
        ],
        compiler_params=pltpu.CompilerParams(collective_id=0),
    )(x2, Wq, K_ext, V_ext, Wo)


# device time: 16445 ns/iter; 1.3186x vs baseline; 1.3186x over previous
import jax
import jax.numpy as jnp
from jax import lax
from jax.experimental import pallas as pl
from jax.experimental.pallas import tpu as pltpu

N_DEV = 4
B, SQ, SKV, DH, D_MODEL = 2, 128, 128, 64, 512
H_LOC = 4
QR = 64


def kernel(x, Wq, K_ext, V_ext, Wo):
    my = lax.axis_index("i")
    K_loc = lax.dynamic_slice_in_dim(K_ext, my * H_LOC, H_LOC, axis=2)
    V_loc = lax.dynamic_slice_in_dim(V_ext, my * H_LOC, H_LOC, axis=2)
    K_loc = jnp.transpose(K_loc, (0, 2, 1, 3))
    V_loc = jnp.transpose(V_loc, (0, 2, 1, 3))
    x2 = x.reshape(B * SQ, D_MODEL)

    def body(x_ref, wq_ref, k_ref, v_ref, wo_ref, out_ref,
             comm_ref, send_sems, recv_sems):
        my_pos = lax.axis_index("i")
        xp = my_pos ^ 1
        yp = 3 - my_pos

        barrier_sem = pltpu.get_barrier_semaphore()
        for nbr in (xp, yp):
            pl.semaphore_signal(
                barrier_sem, inc=1,
                device_id=(nbr,), device_id_type=pl.DeviceIdType.MESH,
            )

        def exchange(src_slot, dst_slot, sem_idx, partner):
            return pltpu.make_async_remote_copy(
                src_ref=comm_ref.at[src_slot],
                dst_ref=comm_ref.at[dst_slot],
                send_sem=send_sems.at[sem_idx],
                recv_sem=recv_sems.at[sem_idx],
                device_id=(partner,),
                device_id_type=pl.DeviceIdType.MESH,
            )

        q2 = jnp.dot(x_ref[:], wq_ref[:],
                     preferred_element_type=jnp.float32)

        stage1 = {}
        stage2 = {}
        for b in range(B):
            ctxs = []
            for h in range(H_LOC):
                qbh = q2[b * SQ:(b + 1) * SQ, h * DH:(h + 1) * DH]
                kbh = k_ref[b, h]
                vbh = v_ref[b, h]
                s = lax.dot_general(
                    qbh, kbh, (((1,), (1,)), ((), ())),
                    preferred_element_type=jnp.float32,
                ) * 0.125
                m = jnp.max(s, axis=-1, keepdims=True)
                e = jnp.exp(s - m)
                w = e / jnp.sum(e, axis=-1, keepdims=True)
                ctxs.append(jnp.dot(w, vbh,
                                    preferred_element_type=jnp.float32))
            ctx_b = jnp.concatenate(ctxs, axis=-1)
            p_b = jnp.dot(ctx_b, wo_ref[:],
                          preferred_element_type=jnp.float32)
            if b == 0:
                pl.semaphore_wait(barrier_sem, 2)
            s1_partner = xp if b == 0 else yp
            for j in range(2):
                qi = 2 * b + j
                comm_ref[4 * qi] = p_b[j * QR:(j + 1) * QR, :]
                r = exchange(4 * qi, 4 * qi + 1, 2 * qi, s1_partner)
                r.start()
                stage1[qi] = r

        for qi in range(4):
            stage1[qi].wait_recv()
            comm_ref[4 * qi + 2] = comm_ref[4 * qi] + comm_ref[4 * qi + 1]
            r = exchange(4 * qi + 2, 4 * qi + 3, 2 * qi + 1,
                         yp if qi < 2 else xp)
            r.start()
            stage2[qi] = r

        for qi in range(4):
            stage2[qi].wait_recv()
            b, j = divmod(qi, 2)
            out_ref[b, j * QR:(j + 1) * QR, :] = (
                comm_ref[4 * qi + 2] + comm_ref[4 * qi + 3]
            )

        for r in list(stage1.values()) + list(stage2.values()):
            r.wait_send()

    return pl.pallas_call(
        body,
        out_shape=jax.ShapeDtypeStruct((B, SQ, D_MODEL), jnp.float32),
        in_specs=[pl.BlockSpec(memory_space=pltpu.VMEM)] * 5,
        out_specs=pl.BlockSpec(memory_space=pltpu.VMEM),
        scratch_shapes=[
            pltpu.VMEM((16, QR, D_MODEL), jnp.float32),
            pltpu.SemaphoreType.DMA((8,)),
            pltpu.SemaphoreType.DMA((8,)),
        ],
        compiler_params=pltpu.CompilerParams(collective_id=0),
    )(x2, Wq, K_loc, V_loc, Wo)


# device time: 15500 ns/iter; 1.3990x vs baseline; 1.0610x over previous
import jax
import jax.numpy as jnp
from jax import lax
from jax.experimental import pallas as pl
from jax.experimental.pallas import tpu as pltpu

N_DEV = 4
B, SQ, SKV, DH, D_MODEL = 2, 128, 128, 64, 512
H_LOC = 4
QR = 64


def kernel(x, Wq, K_ext, V_ext, Wo):
    my = lax.axis_index("i")
    K_loc = lax.dynamic_slice_in_dim(K_ext, my * H_LOC, H_LOC, axis=2)
    V_loc = lax.dynamic_slice_in_dim(V_ext, my * H_LOC, H_LOC, axis=2)
    K_loc = jnp.transpose(K_loc, (0, 2, 1, 3))
    V_loc = jnp.transpose(V_loc, (0, 2, 1, 3))
    x2 = x.reshape(B * SQ, D_MODEL)

    def body(x_ref, wq_ref, k_ref, v_ref, wo_ref, out_ref,
             comm_ref, send_sems, recv_sems, out_sems):
        my_pos = lax.axis_index("i")
        xp = my_pos ^ 1
        yp = 3 - my_pos

        barrier_sem = pltpu.get_barrier_semaphore()
        for nbr in (xp, yp):
            pl.semaphore_signal(
                barrier_sem, inc=1,
                device_id=(nbr,), device_id_type=pl.DeviceIdType.MESH,
            )

        def exchange(src_slot, dst_slot, sem_idx, partner):
            return pltpu.make_async_remote_copy(
                src_ref=comm_ref.at[src_slot],
                dst_ref=comm_ref.at[dst_slot],
                send_sem=send_sems.at[sem_idx],
                recv_sem=recv_sems.at[sem_idx],
                device_id=(partner,),
                device_id_type=pl.DeviceIdType.MESH,
            )

        q2 = jnp.dot(x_ref[:], wq_ref[:],
                     preferred_element_type=jnp.float32) * 0.125

        stage1 = {}
        stage2 = {}
        for b in range(B):
            ss = []
            for h in range(H_LOC):
                qbh = q2[b * SQ:(b + 1) * SQ, h * DH:(h + 1) * DH]
                ss.append(lax.dot_general(
                    qbh, k_ref[b, h], (((1,), (1,)), ((), ())),
                    preferred_element_type=jnp.float32,
                ))
            s_all = jnp.concatenate(ss, axis=0)
            m = jnp.max(s_all, axis=-1, keepdims=True)
            e = jnp.exp(s_all - m)
            w_all = e / jnp.sum(e, axis=-1, keepdims=True)
            ctxs = [
                jnp.dot(w_all[h * SQ:(h + 1) * SQ, :], v_ref[b, h],
                        preferred_element_type=jnp.float32)
                for h in range(H_LOC)
            ]
            ctx_b = jnp.concatenate(ctxs, axis=-1)
            p_b = jnp.dot(ctx_b, wo_ref[:],
                          preferred_element_type=jnp.float32)
            if b == 0:
                pl.semaphore_wait(barrier_sem, 2)
            s1_partner = xp if b == 0 else yp
            for j in range(2):
                qi = 2 * b + j
                comm_ref[4 * qi] = p_b[j * QR:(j + 1) * QR, :]
                r = exchange(4 * qi, 4 * qi + 1, 2 * qi, s1_partner)
                r.start()
                stage1[qi] = r

        for qi in range(4):
            stage1[qi].wait_recv()
            comm_ref[4 * qi + 2] = comm_ref[4 * qi] + comm_ref[4 * qi + 1]
            r = exchange(4 * qi + 2, 4 * qi + 3, 2 * qi + 1,
                         yp if qi < 2 else xp)
            r.start()
            stage2[qi] = r

        out_copies = []
        for qi in range(4):
            stage2[qi].wait_recv()
            b, j = divmod(qi, 2)
            comm_ref[16 + qi] = comm_ref[4 * qi + 2] + comm_ref[4 * qi + 3]
            c = pltpu.make_async_copy(
                comm_ref.at[16 + qi],
                out_ref.at[b, pl.ds(j * QR, QR), :],
                out_sems.at[qi],
            )
            c.start()
            out_copies.append(c)

        for c in out_copies:
            c.wait()
        for r in list(stage1.values()) + list(stage2.values()):
            r.wait_send()

    return pl.pallas_call(
        body,
        out_shape=jax.ShapeDtypeStruct((B, SQ, D_MODEL), jnp.float32),
        in_specs=[pl.BlockSpec(memory_space=pltpu.VMEM)] * 5,
        out_specs=pl.BlockSpec(memory_space=pltpu.MemorySpace.HBM),
        scratch_shapes=[
            pltpu.VMEM((20, QR, D_MODEL), jnp.float32),
            pltpu.SemaphoreType.DMA((8,)),
            pltpu.SemaphoreType.DMA((8,)),
            pltpu.SemaphoreType.DMA((4,)),
        ],
        compiler_params=pltpu.CompilerParams(collective_id=0),
    )(x2, Wq, K_loc, V_loc, Wo)


# device time: 13287 ns/iter; 1.6320x vs baseline; 1.1666x over previous
import jax
import jax.numpy as jnp
from jax import lax
from jax.experimental import pallas as pl
from jax.experimental.pallas import tpu as pltpu

N_DEV = 4
B, SQ, SKV, DH, D_MODEL = 2, 128, 128, 64, 512
H_LOC = 4
QR = 64


def kernel(x, Wq, K_ext, V_ext, Wo):
    my = lax.axis_index("i")
    K_loc = lax.dynamic_slice_in_dim(K_ext, my * H_LOC, H_LOC, axis=2)
    V_loc = lax.dynamic_slice_in_dim(V_ext, my * H_LOC, H_LOC, axis=2)
    K_loc = jnp.transpose(K_loc, (0, 2, 1, 3))
    V_loc = jnp.transpose(V_loc, (0, 2, 1, 3))
    x2 = x.reshape(B * SQ, D_MODEL)

    def body(x_ref, wq_ref, k_ref, v_ref, wo_ref, out_ref,
             comm_ref, fsum_ref, send_sems, recv_sems, out_sems):
        my_pos = lax.axis_index("i")
        xp = my_pos ^ 1
        yp = 3 - my_pos

        barrier_sem = pltpu.get_barrier_semaphore()
        for nbr in (xp, yp):
            pl.semaphore_signal(
                barrier_sem, inc=1,
                device_id=(nbr,), device_id_type=pl.DeviceIdType.MESH,
            )

        def exchange(src_slot, dst_slot, sem_idx, partner):
            return pltpu.make_async_remote_copy(
                src_ref=comm_ref.at[src_slot],
                dst_ref=comm_ref.at[dst_slot],
                send_sem=send_sems.at[sem_idx],
                recv_sem=recv_sems.at[sem_idx],
                device_id=(partner,),
                device_id_type=pl.DeviceIdType.MESH,
            )

        q2 = jnp.dot(x_ref[:], wq_ref[:],
                     preferred_element_type=jnp.float32) * 0.125

        stage1 = {}
        stage2 = {}
        for b in range(B):
            ss = []
            for h in range(H_LOC):
                qbh = q2[b * SQ:(b + 1) * SQ, h * DH:(h + 1) * DH]
                ss.append(lax.dot_general(
                    qbh, k_ref[b, h], (((1,), (1,)), ((), ())),
                    preferred_element_type=jnp.float32,
                ))
            s_all = jnp.concatenate(ss, axis=0)
            m = jnp.max(s_all, axis=-1, keepdims=True)
            e = jnp.exp(s_all - m)
            w_all = e / jnp.sum(e, axis=-1, keepdims=True)
            ctxs = [
                jnp.dot(w_all[h * SQ:(h + 1) * SQ, :], v_ref[b, h],
                        preferred_element_type=jnp.float32)
                for h in range(H_LOC)
            ]
            ctx_b = jnp.concatenate(ctxs, axis=-1)
            p_b = jnp.dot(ctx_b, wo_ref[:],
                          preferred_element_type=jnp.float32)
            if b == 0:
                pl.semaphore_wait(barrier_sem, 2)
            s1_partner = xp if b == 0 else yp
            for j in range(2):
                qi = 2 * b + j
                comm_ref[4 * qi] = p_b[j * QR:(j + 1) * QR, :].astype(
                    jnp.bfloat16)
                r = exchange(4 * qi, 4 * qi + 1, 2 * qi, s1_partner)
                r.start()
                stage1[qi] = r

        for qi in range(4):
            stage1[qi].wait_recv()
            comm_ref[4 * qi + 2] = (
                comm_ref[4 * qi].astype(jnp.float32)
                + comm_ref[4 * qi + 1].astype(jnp.float32)
            ).astype(jnp.bfloat16)
            r = exchange(4 * qi + 2, 4 * qi + 3, 2 * qi + 1,
                         yp if qi < 2 else xp)
            r.start()
            stage2[qi] = r

        out_copies = []
        for qi in range(4):
            stage2[qi].wait_recv()
            b, j = divmod(qi, 2)
            fsum_ref[qi] = (
                comm_ref[4 * qi + 2].astype(jnp.float32)
                + comm_ref[4 * qi + 3].astype(jnp.float32)
            )
            c = pltpu.make_async_copy(
                fsum_ref.at[qi],
                out_ref.at[b, pl.ds(j * QR, QR), :],
                out_sems.at[qi],
            )
            c.start()
            out_copies.append(c)

        for c in out_copies:
            c.wait()
        for r in list(stage1.values()) + list(stage2.values()):
            r.wait_send()

    return pl.pallas_call(
        body,
        out_shape=jax.ShapeDtypeStruct((B, SQ, D_MODEL), jnp.float32),
        in_specs=[pl.BlockSpec(memory_space=pltpu.VMEM)] * 5,
        out_specs=pl.BlockSpec(memory_space=pltpu.MemorySpace.HBM),
        scratch_shapes=[
            pltpu.VMEM((16, QR, D_MODEL), jnp.bfloat16),
            pltpu.VMEM((4, QR, D_MODEL), jnp.float32),
            pltpu.SemaphoreType.DMA((8,)),
            pltpu.SemaphoreType.DMA((8,)),
            pltpu.SemaphoreType.DMA((4,)),
        ],
        compiler_params=pltpu.CompilerParams(collective_id=0),
    )(x2, Wq, K_loc, V_loc, Wo)
